# Initial kernel scaffold; baseline (speedup 1.0000x reference)
import jax
import jax.numpy as jnp
from jax.experimental import pallas as pl


def kernel(edge_index, edge_types, patient_masks, direct_features, emb, Wr1, Wroot1, b1, Wr2, Wroot2, b2, attn_w, attn_b, Wd, bd, gamma, beta, Wc1, bc1, Wc2, bc2, Wc3, bc3):
    raise NotImplementedError("write your pallas kernel here")


# speedup vs baseline: 2.0422x; 2.0422x over previous
import functools

import jax
import jax.numpy as jnp
from jax import lax
from jax.experimental import pallas as pl
from jax.experimental.pallas import tpu as pltpu
from jax.experimental.pallas import tpu_sc as plsc

_N = 10000
_NP = 10240
_E = 160000
_R = 8
_D = 256
_HC = 128
_CF = _R * _NP
_CNTF = 81920
_EPS = 10000
_K = 80
_NCHUNK = _EPS // _K
_TOT_CHUNKS = _E // _K

_mesh2 = plsc.VectorSubcoreMesh(
    core_axis_name="c", subcore_axis_name="s", num_cores=2, num_subcores=16)


def _zero_vec_loop(ref, nvec):
  def body(i, _):
    ref[pl.ds(i * 16, 16)] = jnp.zeros((16,), jnp.float32)
    return 0
  lax.fori_loop(0, nvec, body, 0)


def _sc_weights_body(esrc_hbm, edst_hbm, et_hbm, w_hbm, gidx_hbm,
                     dbuf, rbuf, sbuf, fbuf, wbuf, gbuf, onesb, invbuf,
                     cnt_sp):
  c = lax.axis_index("c")
  s = lax.axis_index("s")
  _zero_vec_loop(invbuf.at[pl.ds(0, 5120)], 320)
  pltpu.sync_copy(invbuf.at[pl.ds(0, 5120)], cnt_sp.at[pl.ds(s * 5120, 5120)])
  for j in range(5):
    onesb[pl.ds(j * 16, 16)] = jnp.ones((16,), jnp.float32)
  plsc.subcore_barrier()

  def cnt_chunk(i, _):
    off = s * _EPS + i * _K
    pltpu.sync_copy(edst_hbm.at[pl.ds(off, _K)], dbuf)
    pltpu.sync_copy(et_hbm.at[pl.ds(off, _K)], rbuf)
    for j in range(5):
      sl = pl.ds(j * 16, 16)
      fbuf[sl] = dbuf[sl] * 8 + rbuf[sl]
    pltpu.sync_copy(onesb, cnt_sp.at[fbuf], add=True)
    return 0
  lax.fori_loop(0, _NCHUNK, cnt_chunk, 0)
  plsc.subcore_barrier()

  pltpu.sync_copy(cnt_sp, invbuf)
  def inv_loop(i, _):
    sl = pl.ds(i * 16, 16)
    invbuf[sl] = 1.0 / jnp.maximum(invbuf[sl], 1.0)
    return 0
  lax.fori_loop(0, 5120, inv_loop, 0)

  wid = s * 2 + c
  nk = jnp.where(wid >= 16, _TOT_CHUNKS // 32, _TOT_CHUNKS // 32 + 1)
  def w_chunk(k, _):
    off = (wid + 32 * k) * _K
    pltpu.sync_copy(edst_hbm.at[pl.ds(off, _K)], dbuf)
    pltpu.sync_copy(et_hbm.at[pl.ds(off, _K)], rbuf)
    pltpu.sync_copy(esrc_hbm.at[pl.ds(off, _K)], sbuf)
    for j in range(5):
      sl = pl.ds(j * 16, 16)
      wbuf[sl] = plsc.load_gather(invbuf, [dbuf[sl] * 8 + rbuf[sl]])
      gbuf[sl] = rbuf[sl] * _NP + sbuf[sl]
    pltpu.sync_copy(wbuf, w_hbm.at[pl.ds(off, _K)])
    pltpu.sync_copy(gbuf, gidx_hbm.at[pl.ds(off, _K)])
    return 0
  lax.fori_loop(0, nk, w_chunk, 0)


_sc_weights = pl.kernel(
    _sc_weights_body,
    out_type=[jax.ShapeDtypeStruct((_E,), jnp.float32),
              jax.ShapeDtypeStruct((_E,), jnp.int32)],
    mesh=_mesh2,
    scratch_types=[
        pltpu.VMEM((_K,), jnp.int32),
        pltpu.VMEM((_K,), jnp.int32),
        pltpu.VMEM((_K,), jnp.int32),
        pltpu.VMEM((_K,), jnp.int32),
        pltpu.VMEM((_K,), jnp.float32),
        pltpu.VMEM((_K,), jnp.int32),
        pltpu.VMEM((_K,), jnp.float32),
        pltpu.VMEM((_CNTF,), jnp.float32),
        pltpu.VMEM_SHARED((_CNTF,), jnp.float32),
    ],
    compiler_params=pltpu.CompilerParams(needs_layout_passes=False),
)


def _sc_agg_body(table_hbm, gidx_hbm, edst_hbm, w_hbm, out_hbm,
                 gbuf, dbuf, wbuf, rows, agg_sp, sem):
  c = lax.axis_index("c")
  s = lax.axis_index("s")
  def zrow(i, _):
    for j in range(8):
      rows[i, pl.ds(j * 16, 16)] = jnp.zeros((16,), jnp.float32)
    return 0
  lax.fori_loop(0, _K, zrow, 0)
  for k in range(8):
    pltpu.sync_copy(rows, agg_sp.at[pl.ds(s * 640 + k * _K, _K)])
  plsc.subcore_barrier()

  coff = c * _CF
  def chunk(i, _):
    off = s * _EPS + i * _K
    pltpu.sync_copy(gidx_hbm.at[pl.ds(off, _K)], gbuf)
    pltpu.sync_copy(edst_hbm.at[pl.ds(off, _K)], dbuf)
    pltpu.sync_copy(w_hbm.at[pl.ds(off, _K)], wbuf)
    for j in range(5):
      sl = pl.ds(j * 16, 16)
      gbuf[sl] = gbuf[sl] + coff
    pltpu.async_copy(table_hbm.at[gbuf], rows, sem).wait()
    def scale(e, _):
      wsp = plsc.load_gather(wbuf, [jnp.full((16,), e, jnp.int32)])
      for j in range(8):
        sl = pl.ds(j * 16, 16)
        rows[e, sl] = rows[e, sl] * wsp
      return 0
    lax.fori_loop(0, _K, scale, 0)
    pltpu.sync_copy(rows, agg_sp.at[dbuf], add=True)
    return 0
  lax.fori_loop(0, _NCHUNK, chunk, 0)
  plsc.subcore_barrier()
  pltpu.sync_copy(agg_sp.at[pl.ds(s * 640, 640)],
                  out_hbm.at[c, pl.ds(s * 640, 640)])


_sc_agg = pl.kernel(
    _sc_agg_body,
    out_type=jax.ShapeDtypeStruct((2, _NP, _HC), jnp.float32),
    mesh=_mesh2,
    scratch_types=[
        pltpu.VMEM((_K,), jnp.int32),
        pltpu.VMEM((_K,), jnp.int32),
        pltpu.VMEM((_K,), jnp.float32),
        pltpu.VMEM((_K, _HC), jnp.float32),
        pltpu.VMEM_SHARED((_NP, _HC), jnp.float32),
        pltpu.SemaphoreType.DMA,
    ],
    compiler_params=pltpu.CompilerParams(needs_layout_passes=False),
)


def _transform_body(x_ref, w_ref, o_ref):
  o_ref[...] = jnp.dot(x_ref[...], w_ref[0],
                       preferred_element_type=jnp.float32)


def _transform(x_pad, Wr):
  return pl.pallas_call(
      _transform_body,
      grid=(2, _R, _NP // 1024),
      in_specs=[
          pl.BlockSpec((1024, _D), lambda c, r, t: (t, 0)),
          pl.BlockSpec((1, _D, _HC), lambda c, r, t: (r, 0, c)),
      ],
      out_specs=pl.BlockSpec(
          (1024, _HC), lambda c, r, t: (c * (_CF // 1024) + r * 10 + t, 0)),
      out_shape=jax.ShapeDtypeStruct((2 * _CF, _HC), jnp.float32),
  )(x_pad, Wr)


def _combine1_body(x_ref, wr_ref, a_ref, b_ref, o_ref):
  y = jnp.dot(x_ref[...], wr_ref[...], preferred_element_type=jnp.float32)
  agg = jnp.concatenate([a_ref[0], a_ref[1]], axis=1)
  o_ref[...] = jnp.maximum(y + agg + b_ref[...], 0.0)


def _combine2_body(x_ref, wr_ref, a_ref, b_ref, aw_ref, ab_ref, o_ref, l_ref):
  y = jnp.dot(x_ref[...], wr_ref[...], preferred_element_type=jnp.float32)
  agg = jnp.concatenate([a_ref[0], a_ref[1]], axis=1)
  y = y + agg + b_ref[...]
  o_ref[...] = y
  l_ref[...] = jnp.tanh(
      jnp.dot(y, aw_ref[...], preferred_element_type=jnp.float32)
      + ab_ref[...])


def _combine1(x_pad, Wroot, agg, b_row):
  return pl.pallas_call(
      _combine1_body,
      grid=(_NP // 1024,),
      in_specs=[
          pl.BlockSpec((1024, _D), lambda t: (t, 0)),
          pl.BlockSpec((_D, _D), lambda t: (0, 0)),
          pl.BlockSpec((2, 1024, _HC), lambda t: (0, t, 0)),
          pl.BlockSpec((1, _D), lambda t: (0, 0)),
      ],
      out_specs=pl.BlockSpec((1024, _D), lambda t: (t, 0)),
      out_shape=jax.ShapeDtypeStruct((_NP, _D), jnp.float32),
  )(x_pad, Wroot, agg, b_row)


def _combine2(x_pad, Wroot, agg, b_row, aw_p, ab_p):
  return pl.pallas_call(
      _combine2_body,
      grid=(_NP // 1024,),
      in_specs=[
          pl.BlockSpec((1024, _D), lambda t: (t, 0)),
          pl.BlockSpec((_D, _D), lambda t: (0, 0)),
          pl.BlockSpec((2, 1024, _HC), lambda t: (0, t, 0)),
          pl.BlockSpec((1, _D), lambda t: (0, 0)),
          pl.BlockSpec((_D, _HC), lambda t: (0, 0)),
          pl.BlockSpec((1, _HC), lambda t: (0, 0)),
      ],
      out_specs=[
          pl.BlockSpec((1024, _D), lambda t: (t, 0)),
          pl.BlockSpec((1024, _HC), lambda t: (t, 0)),
      ],
      out_shape=[
          jax.ShapeDtypeStruct((_NP, _D), jnp.float32),
          jax.ShapeDtypeStruct((_NP, _HC), jnp.float32),
      ],
  )(x_pad, Wroot, agg, b_row, aw_p, ab_p)


def _head_body(l_ref, m_ref, ne_ref, df_ref, wd_ref, bd_ref, g_ref, be_ref,
               w1_ref, b1_ref, w2_ref, b2_ref, w3_ref, b3_ref,
               logit_ref, attn_ref):
  lrow = jnp.broadcast_to(l_ref[...], (16, _NP))
  cols = lax.broadcasted_iota(jnp.int32, (16, _NP), 1)
  lg = jnp.where(m_ref[...] == 0, -1e9, lrow)
  lg = jnp.where(cols >= _N, -jnp.inf, lg)
  mx = jnp.max(lg, axis=1, keepdims=True)
  ex = jnp.exp(lg - mx)
  sm = jnp.sum(ex, axis=1, keepdims=True)
  attn = ex / sm
  attn_ref[...] = attn
  pg = jnp.dot(attn, ne_ref[...], preferred_element_type=jnp.float32)
  h = jnp.maximum(
      jnp.dot(df_ref[...], wd_ref[...], preferred_element_type=jnp.float32)
      + bd_ref[...], 0.0)
  h = h / jnp.sqrt(1.0 + 1e-5) * g_ref[...] + be_ref[...]
  comb = jnp.concatenate([pg, h], axis=1)
  z = jnp.maximum(
      jnp.dot(comb, w1_ref[...], preferred_element_type=jnp.float32)
      + b1_ref[...], 0.0)
  z = jnp.maximum(
      jnp.dot(z, w2_ref[...], preferred_element_type=jnp.float32)
      + b2_ref[...], 0.0)
  logit_ref[...] = (
      jnp.dot(z, w3_ref[...], preferred_element_type=jnp.float32)
      + b3_ref[...])


def _head(lrow, mask_pad, ne, df_pad, wd_p, bd_p, g_p, be_p,
          w1_p, b1_p, w2_p, b2_p, w3_p, b3_p):
  return pl.pallas_call(
      _head_body,
      out_shape=[
          jax.ShapeDtypeStruct((16, _HC), jnp.float32),
          jax.ShapeDtypeStruct((16, _NP), jnp.float32),
      ],
  )(lrow, mask_pad, ne, df_pad, wd_p, bd_p, g_p, be_p,
    w1_p, b1_p, w2_p, b2_p, w3_p, b3_p)


@jax.jit
def kernel(edge_index, edge_types, patient_masks, direct_features, emb,
           Wr1, Wroot1, b1, Wr2, Wroot2, b2, attn_w, attn_b, Wd, bd,
           gamma, beta, Wc1, bc1, Wc2, bc2, Wc3, bc3):
  esrc = edge_index[0]
  edst = edge_index[1]
  w_e, gidx = _sc_weights(esrc, edst, edge_types)

  emb_pad = jnp.pad(emb, ((0, _NP - _N), (0, 0)))
  t1 = _transform(emb_pad, Wr1)
  agg1 = _sc_agg(t1, gidx, edst, w_e)
  x1 = _combine1(emb_pad, Wroot1, agg1, b1.reshape(1, -1))

  aw_p = jnp.pad(attn_w, ((0, 0), (0, _HC - 1)))
  ab_p = jnp.pad(attn_b.reshape(1, 1), ((0, 0), (0, _HC - 1)))
  t2 = _transform(x1, Wr2)
  agg2 = _sc_agg(t2, gidx, edst, w_e)
  ne, ln = _combine2(x1, Wroot2, agg2, b2.reshape(1, -1), aw_p, ab_p)

  lrow = ln[:, 0].reshape(1, _NP)
  mask_pad = jnp.pad(patient_masks, ((0, 0), (0, _NP - _N)))
  df_pad = jnp.pad(direct_features, ((0, 0), (0, _HC - 11)))
  wd_p = jnp.pad(Wd, ((0, _HC - 11), (0, _HC - 64)))
  bd_p = jnp.pad(bd.reshape(1, -1), ((0, 0), (0, _HC - 64)))
  g_p = jnp.pad(gamma.reshape(1, -1), ((0, 0), (0, _HC - 64)))
  be_p = jnp.pad(beta.reshape(1, -1), ((0, 0), (0, _HC - 64)))
  w1_p = jnp.zeros((_D + _HC, _HC), jnp.float32)
  w1_p = w1_p.at[:_D, :64].set(Wc1[:_D])
  w1_p = w1_p.at[_D:_D + 64, :64].set(Wc1[_D:])
  b1_p = jnp.pad(bc1.reshape(1, -1), ((0, 0), (0, _HC - 64)))
  w2_p = jnp.pad(Wc2, ((0, _HC - 64), (0, _HC - 32)))
  b2_p = jnp.pad(bc2.reshape(1, -1), ((0, 0), (0, _HC - 32)))
  w3_p = jnp.pad(Wc3, ((0, _HC - 32), (0, _HC - 3)))
  b3_p = jnp.pad(bc3.reshape(1, -1), ((0, 0), (0, _HC - 3)))

  logits_p, attn = _head(lrow, mask_pad, ne, df_pad, wd_p, bd_p, g_p, be_p,
                         w1_p, b1_p, w2_p, b2_p, w3_p, b3_p)
  return logits_p[:, :3], attn[:, :_N]


# speedup vs baseline: 3.9989x; 1.9582x over previous
import functools

import jax
import jax.numpy as jnp
from jax import lax
from jax.experimental import pallas as pl
from jax.experimental.pallas import tpu as pltpu
from jax.experimental.pallas import tpu_sc as plsc

_N = 10000
_NP = 10240
_E = 160000
_R = 8
_D = 256
_HC = 128
_CF = _R * _NP
_CNTF = 81920
_EPS = 10000
_K = 80
_NCHUNK = _EPS // _K
_TOT_CHUNKS = _E // _K

_mesh2 = plsc.VectorSubcoreMesh(
    core_axis_name="c", subcore_axis_name="s", num_cores=2, num_subcores=16)


def _zero_vec_loop(ref, nvec):
  def body(i, _):
    ref[pl.ds(i * 16, 16)] = jnp.zeros((16,), jnp.float32)
    return 0
  lax.fori_loop(0, nvec, body, 0)


def _sc_weights_body(esrc_hbm, edst_hbm, et_hbm, w_hbm, gidx_hbm,
                     dstage, rstage, sstage, wout, fbuf, onesb, invbuf,
                     cnt_sp):
  c = lax.axis_index("c")
  s = lax.axis_index("s")
  _zero_vec_loop(invbuf.at[pl.ds(0, 5120)], 320)
  pltpu.sync_copy(invbuf.at[pl.ds(0, 5120)], cnt_sp.at[pl.ds(s * 5120, 5120)])
  for j in range(5):
    onesb[pl.ds(j * 16, 16)] = jnp.ones((16,), jnp.float32)
  base = s * _EPS
  pltpu.sync_copy(edst_hbm.at[pl.ds(base, _EPS)], dstage)
  pltpu.sync_copy(et_hbm.at[pl.ds(base, _EPS)], rstage)
  pltpu.sync_copy(esrc_hbm.at[pl.ds(base, _EPS)], sstage)
  plsc.subcore_barrier()

  def cnt_chunk(i, _):
    off = i * _K
    for j in range(5):
      dsl = pl.ds(j * 16, 16)
      ssl = pl.ds(off + j * 16, 16)
      r = rstage[ssl]
      f = dstage[ssl] * 8 + r
      fbuf[dsl] = f
      dstage[ssl] = f
      sstage[ssl] = r * _NP + sstage[ssl]
    pltpu.sync_copy(onesb, cnt_sp.at[fbuf], add=True)
    return 0
  lax.fori_loop(0, _NCHUNK, cnt_chunk, 0)
  plsc.subcore_barrier()

  @pl.when(c == 0)
  def _():
    pltpu.sync_copy(cnt_sp, invbuf)
    def inv_loop(i, _):
      sl = pl.ds(i * 16, 16)
      invbuf[sl] = 1.0 / jnp.maximum(invbuf[sl], 1.0)
      return 0
    lax.fori_loop(0, 5120, inv_loop, 0)
    def w_chunk(i, _):
      sl = pl.ds(i * 16, 16)
      wout[sl] = plsc.load_gather(invbuf, [dstage[sl]])
      return 0
    lax.fori_loop(0, _EPS // 16, w_chunk, 0)
    pltpu.sync_copy(wout, w_hbm.at[pl.ds(base, _EPS)])
    pltpu.sync_copy(sstage, gidx_hbm.at[pl.ds(base, _EPS)])


_sc_weights = pl.kernel(
    _sc_weights_body,
    out_type=[jax.ShapeDtypeStruct((_E,), jnp.float32),
              jax.ShapeDtypeStruct((_E,), jnp.int32)],
    mesh=_mesh2,
    scratch_types=[
        pltpu.VMEM((_EPS,), jnp.int32),
        pltpu.VMEM((_EPS,), jnp.int32),
        pltpu.VMEM((_EPS,), jnp.int32),
        pltpu.VMEM((_EPS,), jnp.float32),
        pltpu.VMEM((_K,), jnp.int32),
        pltpu.VMEM((_K,), jnp.float32),
        pltpu.VMEM((_CNTF,), jnp.float32),
        pltpu.VMEM_SHARED((_CNTF,), jnp.float32),
    ],
    compiler_params=pltpu.CompilerParams(needs_layout_passes=False),
)


def _sc_agg_body(table_hbm, gidx_hbm, edst_hbm, w_hbm, out_hbm,
                 gstage, gbufA, dbufA, wbufA, rowsA,
                 gbufB, dbufB, wbufB, rowsB, agg_sp, semA, semB):
  c = lax.axis_index("c")
  s = lax.axis_index("s")
  def zrow(i, _):
    for j in range(8):
      rowsA[i, pl.ds(j * 16, 16)] = jnp.zeros((16,), jnp.float32)
    return 0
  lax.fori_loop(0, _K, zrow, 0)
  for k in range(8):
    pltpu.sync_copy(rowsA, agg_sp.at[pl.ds(s * 640 + k * _K, _K)])
  base = s * _EPS
  pltpu.sync_copy(gidx_hbm.at[pl.ds(base, _EPS)], gstage)
  plsc.subcore_barrier()

  coff = c * _CF

  def prep(off, gbuf, dbuf, wbuf, rows, sem):
    for j in range(5):
      gbuf[pl.ds(j * 16, 16)] = gstage[pl.ds(off + j * 16, 16)] + coff
    pltpu.async_copy(table_hbm.at[gbuf], rows, sem)
    pltpu.async_copy(edst_hbm.at[pl.ds(base + off, _K)], dbuf, sem)
    pltpu.async_copy(w_hbm.at[pl.ds(base + off, _K)], wbuf, sem)

  def proc(off, gbuf, dbuf, wbuf, rows, sem):
    pltpu.make_async_copy(table_hbm.at[gbuf], rows, sem).wait()
    pltpu.make_async_copy(edst_hbm.at[pl.ds(base + off, _K)], dbuf, sem).wait()
    pltpu.make_async_copy(w_hbm.at[pl.ds(base + off, _K)], wbuf, sem).wait()
    def scale(e, _):
      wsp = plsc.load_gather(wbuf, [jnp.full((16,), e, jnp.int32)])
      for j in range(8):
        sl = pl.ds(j * 16, 16)
        rows[e, sl] = rows[e, sl] * wsp
      return 0
    lax.fori_loop(0, _K, scale, 0)
    pltpu.sync_copy(rows, agg_sp.at[dbuf], add=True)

  prep(0, gbufA, dbufA, wbufA, rowsA, semA)
  def pair(k, _):
    o0 = 2 * k * _K
    prep(o0 + _K, gbufB, dbufB, wbufB, rowsB, semB)
    proc(o0, gbufA, dbufA, wbufA, rowsA, semA)
    prep(o0 + 2 * _K, gbufA, dbufA, wbufA, rowsA, semA)
    proc(o0 + _K, gbufB, dbufB, wbufB, rowsB, semB)
    return 0
  lax.fori_loop(0, (_NCHUNK - 1) // 2, pair, 0)
  proc((_NCHUNK - 1) * _K, gbufA, dbufA, wbufA, rowsA, semA)

  plsc.subcore_barrier()
  pltpu.sync_copy(agg_sp.at[pl.ds(s * 640, 640)],
                  out_hbm.at[c, pl.ds(s * 640, 640)])


_sc_agg = pl.kernel(
    _sc_agg_body,
    out_type=jax.ShapeDtypeStruct((2, _NP, _HC), jnp.float32),
    mesh=_mesh2,
    scratch_types=[
        pltpu.VMEM((_EPS,), jnp.int32),
        pltpu.VMEM((_K,), jnp.int32),
        pltpu.VMEM((_K,), jnp.int32),
        pltpu.VMEM((_K,), jnp.float32),
        pltpu.VMEM((_K, _HC), jnp.float32),
        pltpu.VMEM((_K,), jnp.int32),
        pltpu.VMEM((_K,), jnp.int32),
        pltpu.VMEM((_K,), jnp.float32),
        pltpu.VMEM((_K, _HC), jnp.float32),
        pltpu.VMEM_SHARED((_NP, _HC), jnp.float32),
        pltpu.SemaphoreType.DMA,
        pltpu.SemaphoreType.DMA,
    ],
    compiler_params=pltpu.CompilerParams(needs_layout_passes=False),
)


def _transform_body(x_ref, w_ref, o_ref):
  o_ref[...] = jnp.dot(x_ref[...], w_ref[0],
                       preferred_element_type=jnp.float32)


def _transform(x_pad, Wr):
  return pl.pallas_call(
      _transform_body,
      grid=(2, _R, _NP // 1024),
      in_specs=[
          pl.BlockSpec((1024, _D), lambda c, r, t: (t, 0)),
          pl.BlockSpec((1, _D, _HC), lambda c, r, t: (r, 0, c)),
      ],
      out_specs=pl.BlockSpec(
          (1024, _HC), lambda c, r, t: (c * (_CF // 1024) + r * 10 + t, 0)),
      out_shape=jax.ShapeDtypeStruct((2 * _CF, _HC), jnp.float32),
  )(x_pad, Wr)


def _combine1_body(x_ref, wr_ref, a_ref, b_ref, o_ref):
  y = jnp.dot(x_ref[...], wr_ref[...], preferred_element_type=jnp.float32)
  agg = jnp.concatenate([a_ref[0], a_ref[1]], axis=1)
  o_ref[...] = jnp.maximum(y + agg + b_ref[...], 0.0)


def _combine2_body(x_ref, wr_ref, a_ref, b_ref, aw_ref, ab_ref, o_ref, l_ref):
  y = jnp.dot(x_ref[...], wr_ref[...], preferred_element_type=jnp.float32)
  agg = jnp.concatenate([a_ref[0], a_ref[1]], axis=1)
  y = y + agg + b_ref[...]
  o_ref[...] = y
  l_ref[...] = jnp.tanh(
      jnp.dot(y, aw_ref[...], preferred_element_type=jnp.float32)
      + ab_ref[...])


def _combine1(x_pad, Wroot, agg, b_row):
  return pl.pallas_call(
      _combine1_body,
      grid=(_NP // 1024,),
      in_specs=[
          pl.BlockSpec((1024, _D), lambda t: (t, 0)),
          pl.BlockSpec((_D, _D), lambda t: (0, 0)),
          pl.BlockSpec((2, 1024, _HC), lambda t: (0, t, 0)),
          pl.BlockSpec((1, _D), lambda t: (0, 0)),
      ],
      out_specs=pl.BlockSpec((1024, _D), lambda t: (t, 0)),
      out_shape=jax.ShapeDtypeStruct((_NP, _D), jnp.float32),
  )(x_pad, Wroot, agg, b_row)


def _combine2(x_pad, Wroot, agg, b_row, aw_p, ab_p):
  return pl.pallas_call(
      _combine2_body,
      grid=(_NP // 1024,),
      in_specs=[
          pl.BlockSpec((1024, _D), lambda t: (t, 0)),
          pl.BlockSpec((_D, _D), lambda t: (0, 0)),
          pl.BlockSpec((2, 1024, _HC), lambda t: (0, t, 0)),
          pl.BlockSpec((1, _D), lambda t: (0, 0)),
          pl.BlockSpec((_D, _HC), lambda t: (0, 0)),
          pl.BlockSpec((1, _HC), lambda t: (0, 0)),
      ],
      out_specs=[
          pl.BlockSpec((1024, _D), lambda t: (t, 0)),
          pl.BlockSpec((1024, _HC), lambda t: (t, 0)),
      ],
      out_shape=[
          jax.ShapeDtypeStruct((_NP, _D), jnp.float32),
          jax.ShapeDtypeStruct((_NP, _HC), jnp.float32),
      ],
  )(x_pad, Wroot, agg, b_row, aw_p, ab_p)


def _head_body(l_ref, m_ref, ne_ref, df_ref, wd_ref, bd_ref, g_ref, be_ref,
               w1_ref, b1_ref, w2_ref, b2_ref, w3_ref, b3_ref,
               logit_ref, attn_ref):
  lrow = jnp.broadcast_to(l_ref[...], (16, _NP))
  cols = lax.broadcasted_iota(jnp.int32, (16, _NP), 1)
  lg = jnp.where(m_ref[...] == 0, -1e9, lrow)
  lg = jnp.where(cols >= _N, -jnp.inf, lg)
  mx = jnp.max(lg, axis=1, keepdims=True)
  ex = jnp.exp(lg - mx)
  sm = jnp.sum(ex, axis=1, keepdims=True)
  attn = ex / sm
  attn_ref[...] = attn
  pg = jnp.dot(attn, ne_ref[...], preferred_element_type=jnp.float32)
  h = jnp.maximum(
      jnp.dot(df_ref[...], wd_ref[...], preferred_element_type=jnp.float32)
      + bd_ref[...], 0.0)
  h = h / jnp.sqrt(1.0 + 1e-5) * g_ref[...] + be_ref[...]
  comb = jnp.concatenate([pg, h], axis=1)
  z = jnp.maximum(
      jnp.dot(comb, w1_ref[...], preferred_element_type=jnp.float32)
      + b1_ref[...], 0.0)
  z = jnp.maximum(
      jnp.dot(z, w2_ref[...], preferred_element_type=jnp.float32)
      + b2_ref[...], 0.0)
  logit_ref[...] = (
      jnp.dot(z, w3_ref[...], preferred_element_type=jnp.float32)
      + b3_ref[...])


def _head(lrow, mask_pad, ne, df_pad, wd_p, bd_p, g_p, be_p,
          w1_p, b1_p, w2_p, b2_p, w3_p, b3_p):
  return pl.pallas_call(
      _head_body,
      out_shape=[
          jax.ShapeDtypeStruct((16, _HC), jnp.float32),
          jax.ShapeDtypeStruct((16, _NP), jnp.float32),
      ],
  )(lrow, mask_pad, ne, df_pad, wd_p, bd_p, g_p, be_p,
    w1_p, b1_p, w2_p, b2_p, w3_p, b3_p)


@jax.jit
def kernel(edge_index, edge_types, patient_masks, direct_features, emb,
           Wr1, Wroot1, b1, Wr2, Wroot2, b2, attn_w, attn_b, Wd, bd,
           gamma, beta, Wc1, bc1, Wc2, bc2, Wc3, bc3):
  esrc = edge_index[0]
  edst = edge_index[1]
  w_e, gidx = _sc_weights(esrc, edst, edge_types)

  emb_pad = jnp.pad(emb, ((0, _NP - _N), (0, 0)))
  t1 = _transform(emb_pad, Wr1)
  agg1 = _sc_agg(t1, gidx, edst, w_e)
  x1 = _combine1(emb_pad, Wroot1, agg1, b1.reshape(1, -1))

  aw_p = jnp.pad(attn_w, ((0, 0), (0, _HC - 1)))
  ab_p = jnp.pad(attn_b.reshape(1, 1), ((0, 0), (0, _HC - 1)))
  t2 = _transform(x1, Wr2)
  agg2 = _sc_agg(t2, gidx, edst, w_e)
  ne, ln = _combine2(x1, Wroot2, agg2, b2.reshape(1, -1), aw_p, ab_p)

  lrow = ln[:, 0].reshape(1, _NP)
  mask_pad = jnp.pad(patient_masks, ((0, 0), (0, _NP - _N)))
  df_pad = jnp.pad(direct_features, ((0, 0), (0, _HC - 11)))
  wd_p = jnp.pad(Wd, ((0, _HC - 11), (0, _HC - 64)))
  bd_p = jnp.pad(bd.reshape(1, -1), ((0, 0), (0, _HC - 64)))
  g_p = jnp.pad(gamma.reshape(1, -1), ((0, 0), (0, _HC - 64)))
  be_p = jnp.pad(beta.reshape(1, -1), ((0, 0), (0, _HC - 64)))
  w1_p = jnp.zeros((_D + _HC, _HC), jnp.float32)
  w1_p = w1_p.at[:_D, :64].set(Wc1[:_D])
  w1_p = w1_p.at[_D:_D + 64, :64].set(Wc1[_D:])
  b1_p = jnp.pad(bc1.reshape(1, -1), ((0, 0), (0, _HC - 64)))
  w2_p = jnp.pad(Wc2, ((0, _HC - 64), (0, _HC - 32)))
  b2_p = jnp.pad(bc2.reshape(1, -1), ((0, 0), (0, _HC - 32)))
  w3_p = jnp.pad(Wc3, ((0, _HC - 32), (0, _HC - 3)))
  b3_p = jnp.pad(bc3.reshape(1, -1), ((0, 0), (0, _HC - 3)))

  logits_p, attn = _head(lrow, mask_pad, ne, df_pad, wd_p, bd_p, g_p, be_p,
                         w1_p, b1_p, w2_p, b2_p, w3_p, b3_p)
  return logits_p[:, :3], attn[:, :_N]


# speedup vs baseline: 4.6180x; 1.1548x over previous
import functools

import jax
import jax.numpy as jnp
from jax import lax
from jax.experimental import pallas as pl
from jax.experimental.pallas import tpu as pltpu
from jax.experimental.pallas import tpu_sc as plsc

_N = 10000
_NP = 10240
_E = 160000
_R = 8
_D = 256
_HC = 128
_CF = _R * _NP
_CNTF = 81920
_EPS = 10000
_K = 80
_NCHUNK = _EPS // _K
_TOT_CHUNKS = _E // _K

_mesh2 = plsc.VectorSubcoreMesh(
    core_axis_name="c", subcore_axis_name="s", num_cores=2, num_subcores=16)


def _zero_vec_loop(ref, nvec):
  def body(i, _):
    ref[pl.ds(i * 16, 16)] = jnp.zeros((16,), jnp.float32)
    return 0
  lax.fori_loop(0, nvec, body, 0)


def _sc_weights_body(esrc_hbm, edst_hbm, et_hbm, w_hbm, gidx_hbm,
                     dstage, rstage, sstage, wout, fbuf, onesb, invbuf,
                     cnt_sp):
  c = lax.axis_index("c")
  s = lax.axis_index("s")
  _zero_vec_loop(invbuf.at[pl.ds(0, 5120)], 320)
  pltpu.sync_copy(invbuf.at[pl.ds(0, 5120)], cnt_sp.at[pl.ds(s * 5120, 5120)])
  for j in range(5):
    onesb[pl.ds(j * 16, 16)] = jnp.ones((16,), jnp.float32)
  base = s * _EPS
  pltpu.sync_copy(edst_hbm.at[pl.ds(base, _EPS)], dstage)
  pltpu.sync_copy(et_hbm.at[pl.ds(base, _EPS)], rstage)
  pltpu.sync_copy(esrc_hbm.at[pl.ds(base, _EPS)], sstage)
  plsc.subcore_barrier()

  def cnt_chunk(i, _):
    off = i * _K
    for j in range(5):
      dsl = pl.ds(j * 16, 16)
      ssl = pl.ds(off + j * 16, 16)
      r = rstage[ssl]
      f = dstage[ssl] * 8 + r
      fbuf[dsl] = f
      dstage[ssl] = f
      sstage[ssl] = r * _NP + sstage[ssl]
    pltpu.sync_copy(onesb, cnt_sp.at[fbuf], add=True)
    return 0
  lax.fori_loop(0, _NCHUNK, cnt_chunk, 0)
  plsc.subcore_barrier()

  @pl.when(c == 0)
  def _():
    pltpu.sync_copy(cnt_sp, invbuf)
    def inv_loop(i, _):
      sl = pl.ds(i * 16, 16)
      invbuf[sl] = 1.0 / jnp.maximum(invbuf[sl], 1.0)
      return 0
    lax.fori_loop(0, 5120, inv_loop, 0)
    def w_chunk(i, _):
      sl = pl.ds(i * 16, 16)
      wout[sl] = plsc.load_gather(invbuf, [dstage[sl]])
      return 0
    lax.fori_loop(0, _EPS // 16, w_chunk, 0)
    pltpu.sync_copy(wout, w_hbm.at[pl.ds(base, _EPS)])
    pltpu.sync_copy(sstage, gidx_hbm.at[pl.ds(base, _EPS)])


_sc_weights = pl.kernel(
    _sc_weights_body,
    out_type=[jax.ShapeDtypeStruct((_E,), jnp.float32),
              jax.ShapeDtypeStruct((_E,), jnp.int32)],
    mesh=_mesh2,
    scratch_types=[
        pltpu.VMEM((_EPS,), jnp.int32),
        pltpu.VMEM((_EPS,), jnp.int32),
        pltpu.VMEM((_EPS,), jnp.int32),
        pltpu.VMEM((_EPS,), jnp.float32),
        pltpu.VMEM((_K,), jnp.int32),
        pltpu.VMEM((_K,), jnp.float32),
        pltpu.VMEM((_CNTF,), jnp.float32),
        pltpu.VMEM_SHARED((_CNTF,), jnp.float32),
    ],
    compiler_params=pltpu.CompilerParams(needs_layout_passes=False),
)


def _sc_agg_body(table_hbm, gidx_hbm, edst_hbm, w_hbm, out_hbm,
                 gstage, gbufA, dbufA, wbufA, rowsA,
                 gbufB, dbufB, wbufB, rowsB,
                 gbufC, dbufC, wbufC, rowsC,
                 agg_sp, gsemA, gsemB, gsemC, ssemA, ssemB, ssemC):
  c = lax.axis_index("c")
  s = lax.axis_index("s")
  def zrow(i, _):
    for j in range(8):
      rowsA[i, pl.ds(j * 16, 16)] = jnp.zeros((16,), jnp.float32)
    return 0
  lax.fori_loop(0, _K, zrow, 0)
  for k in range(8):
    pltpu.sync_copy(rowsA, agg_sp.at[pl.ds(s * 640 + k * _K, _K)])
  base = s * _EPS
  pltpu.sync_copy(gidx_hbm.at[pl.ds(base, _EPS)], gstage)
  plsc.subcore_barrier()

  coff = c * _CF
  bufs = ((gbufA, dbufA, wbufA, rowsA, gsemA, ssemA),
          (gbufB, dbufB, wbufB, rowsB, gsemB, ssemB),
          (gbufC, dbufC, wbufC, rowsC, gsemC, ssemC))

  def prep(off, b):
    gbuf, dbuf, wbuf, rows, gsem, _ = b
    for j in range(5):
      gbuf[pl.ds(j * 16, 16)] = gstage[pl.ds(off + j * 16, 16)] + coff
    pltpu.async_copy(table_hbm.at[gbuf], rows, gsem)
    pltpu.async_copy(edst_hbm.at[pl.ds(base + off, _K)], dbuf, gsem)
    pltpu.async_copy(w_hbm.at[pl.ds(base + off, _K)], wbuf, gsem)

  def scale_scatter(b):
    gbuf, dbuf, wbuf, rows, gsem, ssem = b
    pltpu.make_async_copy(table_hbm.at[gbuf], rows, gsem).wait()
    pltpu.make_async_copy(edst_hbm.at[pl.ds(base, _K)], dbuf, gsem).wait()
    pltpu.make_async_copy(w_hbm.at[pl.ds(base, _K)], wbuf, gsem).wait()
    @plsc.parallel_loop(0, _K, unroll=2)
    def scale(e):
      wsp = plsc.load_gather(wbuf, [jnp.full((16,), e, jnp.int32)])
      for j in range(8):
        sl = pl.ds(j * 16, 16)
        rows[e, sl] = rows[e, sl] * wsp
    pltpu.async_copy(rows, agg_sp.at[dbuf], ssem, add=True)

  def wait_scatter(b):
    _, dbuf, _, rows, _, ssem = b
    pltpu.make_async_copy(rows, agg_sp.at[dbuf], ssem).wait()

  prep(0, bufs[0])
  prep(_K, bufs[1])
  prep(2 * _K, bufs[2])
  def ring(k, _):
    o = 3 * k * _K
    scale_scatter(bufs[0])
    scale_scatter(bufs[1])
    wait_scatter(bufs[0])
    prep(o + 3 * _K, bufs[0])
    scale_scatter(bufs[2])
    wait_scatter(bufs[1])
    prep(o + 4 * _K, bufs[1])
    wait_scatter(bufs[2])
    prep(o + 5 * _K, bufs[2])
    return 0
  lax.fori_loop(0, 40, ring, 0)
  scale_scatter(bufs[0])
  scale_scatter(bufs[1])
  wait_scatter(bufs[0])
  prep(123 * _K, bufs[0])
  scale_scatter(bufs[2])
  wait_scatter(bufs[1])
  prep(124 * _K, bufs[1])
  scale_scatter(bufs[0])
  scale_scatter(bufs[1])
  wait_scatter(bufs[2])
  wait_scatter(bufs[0])
  wait_scatter(bufs[1])

  plsc.subcore_barrier()
  pltpu.sync_copy(agg_sp.at[pl.ds(s * 640, 640)],
                  out_hbm.at[c, pl.ds(s * 640, 640)])


_sc_agg = pl.kernel(
    _sc_agg_body,
    out_type=jax.ShapeDtypeStruct((2, _NP, _HC), jnp.float32),
    mesh=_mesh2,
    scratch_types=[
        pltpu.VMEM((_EPS,), jnp.int32),
        pltpu.VMEM((_K,), jnp.int32),
        pltpu.VMEM((_K,), jnp.int32),
        pltpu.VMEM((_K,), jnp.float32),
        pltpu.VMEM((_K, _HC), jnp.float32),
        pltpu.VMEM((_K,), jnp.int32),
        pltpu.VMEM((_K,), jnp.int32),
        pltpu.VMEM((_K,), jnp.float32),
        pltpu.VMEM((_K, _HC), jnp.float32),
        pltpu.VMEM((_K,), jnp.int32),
        pltpu.VMEM((_K,), jnp.int32),
        pltpu.VMEM((_K,), jnp.float32),
        pltpu.VMEM((_K, _HC), jnp.float32),
        pltpu.VMEM_SHARED((_NP, _HC), jnp.float32),
        pltpu.SemaphoreType.DMA,
        pltpu.SemaphoreType.DMA,
        pltpu.SemaphoreType.DMA,
        pltpu.SemaphoreType.DMA,
        pltpu.SemaphoreType.DMA,
        pltpu.SemaphoreType.DMA,
    ],
    compiler_params=pltpu.CompilerParams(needs_layout_passes=False),
)


def _transform_body(x_ref, w_ref, o_ref):
  o_ref[...] = jnp.dot(x_ref[...], w_ref[0],
                       preferred_element_type=jnp.float32)


def _transform(x_pad, Wr):
  return pl.pallas_call(
      _transform_body,
      grid=(2, _R, _NP // 1024),
      in_specs=[
          pl.BlockSpec((1024, _D), lambda c, r, t: (t, 0)),
          pl.BlockSpec((1, _D, _HC), lambda c, r, t: (r, 0, c)),
      ],
      out_specs=pl.BlockSpec(
          (1024, _HC), lambda c, r, t: (c * (_CF // 1024) + r * 10 + t, 0)),
      out_shape=jax.ShapeDtypeStruct((2 * _CF, _HC), jnp.float32),
  )(x_pad, Wr)


def _combine1_body(x_ref, wr_ref, a_ref, b_ref, o_ref):
  y = jnp.dot(x_ref[...], wr_ref[...], preferred_element_type=jnp.float32)
  agg = jnp.concatenate([a_ref[0], a_ref[1]], axis=1)
  o_ref[...] = jnp.maximum(y + agg + b_ref[...], 0.0)


def _combine2_body(x_ref, wr_ref, a_ref, b_ref, aw_ref, ab_ref, o_ref, l_ref):
  y = jnp.dot(x_ref[...], wr_ref[...], preferred_element_type=jnp.float32)
  agg = jnp.concatenate([a_ref[0], a_ref[1]], axis=1)
  y = y + agg + b_ref[...]
  o_ref[...] = y
  l_ref[...] = jnp.tanh(
      jnp.dot(y, aw_ref[...], preferred_element_type=jnp.float32)
      + ab_ref[...])


def _combine1(x_pad, Wroot, agg, b_row):
  return pl.pallas_call(
      _combine1_body,
      grid=(_NP // 1024,),
      in_specs=[
          pl.BlockSpec((1024, _D), lambda t: (t, 0)),
          pl.BlockSpec((_D, _D), lambda t: (0, 0)),
          pl.BlockSpec((2, 1024, _HC), lambda t: (0, t, 0)),
          pl.BlockSpec((1, _D), lambda t: (0, 0)),
      ],
      out_specs=pl.BlockSpec((1024, _D), lambda t: (t, 0)),
      out_shape=jax.ShapeDtypeStruct((_NP, _D), jnp.float32),
  )(x_pad, Wroot, agg, b_row)


def _combine2(x_pad, Wroot, agg, b_row, aw_p, ab_p):
  return pl.pallas_call(
      _combine2_body,
      grid=(_NP // 1024,),
      in_specs=[
          pl.BlockSpec((1024, _D), lambda t: (t, 0)),
          pl.BlockSpec((_D, _D), lambda t: (0, 0)),
          pl.BlockSpec((2, 1024, _HC), lambda t: (0, t, 0)),
          pl.BlockSpec((1, _D), lambda t: (0, 0)),
          pl.BlockSpec((_D, _HC), lambda t: (0, 0)),
          pl.BlockSpec((1, _HC), lambda t: (0, 0)),
      ],
      out_specs=[
          pl.BlockSpec((1024, _D), lambda t: (t, 0)),
          pl.BlockSpec((1024, _HC), lambda t: (t, 0)),
      ],
      out_shape=[
          jax.ShapeDtypeStruct((_NP, _D), jnp.float32),
          jax.ShapeDtypeStruct((_NP, _HC), jnp.float32),
      ],
  )(x_pad, Wroot, agg, b_row, aw_p, ab_p)


def _head_body(l_ref, m_ref, ne_ref, df_ref, wd_ref, bd_ref, g_ref, be_ref,
               w1_ref, b1_ref, w2_ref, b2_ref, w3_ref, b3_ref,
               logit_ref, attn_ref):
  lrow = jnp.broadcast_to(l_ref[...], (16, _NP))
  cols = lax.broadcasted_iota(jnp.int32, (16, _NP), 1)
  lg = jnp.where(m_ref[...] == 0, -1e9, lrow)
  lg = jnp.where(cols >= _N, -jnp.inf, lg)
  mx = jnp.max(lg, axis=1, keepdims=True)
  ex = jnp.exp(lg - mx)
  sm = jnp.sum(ex, axis=1, keepdims=True)
  attn = ex / sm
  attn_ref[...] = attn
  pg = jnp.dot(attn, ne_ref[...], preferred_element_type=jnp.float32)
  h = jnp.maximum(
      jnp.dot(df_ref[...], wd_ref[...], preferred_element_type=jnp.float32)
      + bd_ref[...], 0.0)
  h = h / jnp.sqrt(1.0 + 1e-5) * g_ref[...] + be_ref[...]
  comb = jnp.concatenate([pg, h], axis=1)
  z = jnp.maximum(
      jnp.dot(comb, w1_ref[...], preferred_element_type=jnp.float32)
      + b1_ref[...], 0.0)
  z = jnp.maximum(
      jnp.dot(z, w2_ref[...], preferred_element_type=jnp.float32)
      + b2_ref[...], 0.0)
  logit_ref[...] = (
      jnp.dot(z, w3_ref[...], preferred_element_type=jnp.float32)
      + b3_ref[...])


def _head(lrow, mask_pad, ne, df_pad, wd_p, bd_p, g_p, be_p,
          w1_p, b1_p, w2_p, b2_p, w3_p, b3_p):
  return pl.pallas_call(
      _head_body,
      out_shape=[
          jax.ShapeDtypeStruct((16, _HC), jnp.float32),
          jax.ShapeDtypeStruct((16, _NP), jnp.float32),
      ],
  )(lrow, mask_pad, ne, df_pad, wd_p, bd_p, g_p, be_p,
    w1_p, b1_p, w2_p, b2_p, w3_p, b3_p)


@jax.jit
def kernel(edge_index, edge_types, patient_masks, direct_features, emb,
           Wr1, Wroot1, b1, Wr2, Wroot2, b2, attn_w, attn_b, Wd, bd,
           gamma, beta, Wc1, bc1, Wc2, bc2, Wc3, bc3):
  esrc = edge_index[0]
  edst = edge_index[1]
  w_e, gidx = _sc_weights(esrc, edst, edge_types)

  emb_pad = jnp.pad(emb, ((0, _NP - _N), (0, 0)))
  t1 = _transform(emb_pad, Wr1)
  agg1 = _sc_agg(t1, gidx, edst, w_e)
  x1 = _combine1(emb_pad, Wroot1, agg1, b1.reshape(1, -1))

  aw_p = jnp.pad(attn_w, ((0, 0), (0, _HC - 1)))
  ab_p = jnp.pad(attn_b.reshape(1, 1), ((0, 0), (0, _HC - 1)))
  t2 = _transform(x1, Wr2)
  agg2 = _sc_agg(t2, gidx, edst, w_e)
  ne, ln = _combine2(x1, Wroot2, agg2, b2.reshape(1, -1), aw_p, ab_p)

  lrow = ln[:, 0].reshape(1, _NP)
  mask_pad = jnp.pad(patient_masks, ((0, 0), (0, _NP - _N)))
  df_pad = jnp.pad(direct_features, ((0, 0), (0, _HC - 11)))
  wd_p = jnp.pad(Wd, ((0, _HC - 11), (0, _HC - 64)))
  bd_p = jnp.pad(bd.reshape(1, -1), ((0, 0), (0, _HC - 64)))
  g_p = jnp.pad(gamma.reshape(1, -1), ((0, 0), (0, _HC - 64)))
  be_p = jnp.pad(beta.reshape(1, -1), ((0, 0), (0, _HC - 64)))
  w1_p = jnp.zeros((_D + _HC, _HC), jnp.float32)
  w1_p = w1_p.at[:_D, :64].set(Wc1[:_D])
  w1_p = w1_p.at[_D:_D + 64, :64].set(Wc1[_D:])
  b1_p = jnp.pad(bc1.reshape(1, -1), ((0, 0), (0, _HC - 64)))
  w2_p = jnp.pad(Wc2, ((0, _HC - 64), (0, _HC - 32)))
  b2_p = jnp.pad(bc2.reshape(1, -1), ((0, 0), (0, _HC - 32)))
  w3_p = jnp.pad(Wc3, ((0, _HC - 32), (0, _HC - 3)))
  b3_p = jnp.pad(bc3.reshape(1, -1), ((0, 0), (0, _HC - 3)))

  logits_p, attn = _head(lrow, mask_pad, ne, df_pad, wd_p, bd_p, g_p, be_p,
                         w1_p, b1_p, w2_p, b2_p, w3_p, b3_p)
  return logits_p[:, :3], attn[:, :_N]


# speedup vs baseline: 5.1055x; 1.1056x over previous
import functools

import jax
import jax.numpy as jnp
from jax import lax
from jax.experimental import pallas as pl
from jax.experimental.pallas import tpu as pltpu
from jax.experimental.pallas import tpu_sc as plsc

_N = 10000
_NP = 10240
_E = 160000
_R = 8
_D = 256
_HC = 128
_CF = _R * _NP
_CNTF = 81920
_EPS = 10000
_K = 80
_NCHUNK = _EPS // _K
_TOT_CHUNKS = _E // _K

_mesh2 = plsc.VectorSubcoreMesh(
    core_axis_name="c", subcore_axis_name="s", num_cores=2, num_subcores=16)


def _zero_vec_loop(ref, nvec):
  def body(i, _):
    ref[pl.ds(i * 16, 16)] = jnp.zeros((16,), jnp.float32)
    return 0
  lax.fori_loop(0, nvec, body, 0)


def _sc_weights_body(esrc_hbm, edst_hbm, et_hbm, w_hbm, gidx_hbm,
                     dstage, rstage, sstage, wout, fbuf, fbuf2, onesb, invbuf,
                     cnt_sp, csemA, csemB):
  c = lax.axis_index("c")
  s = lax.axis_index("s")
  _zero_vec_loop(invbuf.at[pl.ds(0, 5120)], 320)
  pltpu.sync_copy(invbuf.at[pl.ds(0, 5120)], cnt_sp.at[pl.ds(s * 5120, 5120)])
  for j in range(5):
    onesb[pl.ds(j * 16, 16)] = jnp.ones((16,), jnp.float32)
  base = s * _EPS
  pltpu.sync_copy(edst_hbm.at[pl.ds(base, _EPS)], dstage)
  pltpu.sync_copy(et_hbm.at[pl.ds(base, _EPS)], rstage)
  pltpu.sync_copy(esrc_hbm.at[pl.ds(base, _EPS)], sstage)
  plsc.subcore_barrier()

  def build(off, fb):
    for j in range(5):
      dsl = pl.ds(j * 16, 16)
      ssl = pl.ds(off + j * 16, 16)
      r = rstage[ssl]
      f = dstage[ssl] * 8 + r
      fb[dsl] = f
      dstage[ssl] = f
      sstage[ssl] = r * _NP + sstage[ssl]

  def cnt_pair(k, _):
    off = 2 * k * _K
    build(off, fbuf)
    pltpu.async_copy(onesb, cnt_sp.at[fbuf], csemA, add=True)
    build(off + _K, fbuf2)
    pltpu.async_copy(onesb, cnt_sp.at[fbuf2], csemB, add=True)
    pltpu.make_async_copy(onesb, cnt_sp.at[fbuf], csemA).wait()
    pltpu.make_async_copy(onesb, cnt_sp.at[fbuf2], csemB).wait()
    return 0
  lax.fori_loop(0, (_NCHUNK - 1) // 2, cnt_pair, 0)
  build((_NCHUNK - 1) * _K, fbuf)
  pltpu.sync_copy(onesb, cnt_sp.at[fbuf], add=True)
  plsc.subcore_barrier()

  @pl.when(c == 0)
  def _():
    pltpu.sync_copy(cnt_sp, invbuf)
    @plsc.parallel_loop(0, 5120, unroll=4)
    def inv_loop(i):
      sl = pl.ds(i * 16, 16)
      invbuf[sl] = 1.0 / jnp.maximum(invbuf[sl], 1.0)
    @plsc.parallel_loop(0, _EPS // 16, unroll=2)
    def w_chunk(i):
      sl = pl.ds(i * 16, 16)
      wout[sl] = plsc.load_gather(invbuf, [dstage[sl]])
    pltpu.sync_copy(wout, w_hbm.at[pl.ds(base, _EPS)])
    pltpu.sync_copy(sstage, gidx_hbm.at[pl.ds(base, _EPS)])


_sc_weights = pl.kernel(
    _sc_weights_body,
    out_type=[jax.ShapeDtypeStruct((_E,), jnp.float32),
              jax.ShapeDtypeStruct((_E,), jnp.int32)],
    mesh=_mesh2,
    scratch_types=[
        pltpu.VMEM((_EPS,), jnp.int32),
        pltpu.VMEM((_EPS,), jnp.int32),
        pltpu.VMEM((_EPS,), jnp.int32),
        pltpu.VMEM((_EPS,), jnp.float32),
        pltpu.VMEM((_K,), jnp.int32),
        pltpu.VMEM((_K,), jnp.int32),
        pltpu.VMEM((_K,), jnp.float32),
        pltpu.VMEM((_CNTF,), jnp.float32),
        pltpu.VMEM_SHARED((_CNTF,), jnp.float32),
        pltpu.SemaphoreType.DMA,
        pltpu.SemaphoreType.DMA,
    ],
    compiler_params=pltpu.CompilerParams(needs_layout_passes=False),
)


def _sc_agg_body(table_hbm, gidx_hbm, edst_hbm, w_hbm, out_hbm,
                 gstage, gbufA, dbufA, wbufA, rowsA,
                 gbufB, dbufB, wbufB, rowsB,
                 gbufC, dbufC, wbufC, rowsC,
                 agg_sp, gsemA, gsemB, gsemC, ssemA, ssemB, ssemC):
  c = lax.axis_index("c")
  s = lax.axis_index("s")
  def zrow(i, _):
    for j in range(8):
      rowsA[i, pl.ds(j * 16, 16)] = jnp.zeros((16,), jnp.float32)
    return 0
  lax.fori_loop(0, _K, zrow, 0)
  for k in range(8):
    pltpu.sync_copy(rowsA, agg_sp.at[pl.ds(s * 640 + k * _K, _K)])
  base = s * _EPS
  pltpu.sync_copy(gidx_hbm.at[pl.ds(base, _EPS)], gstage)
  plsc.subcore_barrier()

  coff = c * _CF
  bufs = ((gbufA, dbufA, wbufA, rowsA, gsemA, ssemA),
          (gbufB, dbufB, wbufB, rowsB, gsemB, ssemB),
          (gbufC, dbufC, wbufC, rowsC, gsemC, ssemC))

  def prep(off, b):
    gbuf, dbuf, wbuf, rows, gsem, _ = b
    for j in range(5):
      gbuf[pl.ds(j * 16, 16)] = gstage[pl.ds(off + j * 16, 16)] + coff
    pltpu.async_copy(table_hbm.at[gbuf], rows, gsem)
    pltpu.async_copy(edst_hbm.at[pl.ds(base + off, _K)], dbuf, gsem)
    pltpu.async_copy(w_hbm.at[pl.ds(base + off, _K)], wbuf, gsem)

  def scale_scatter(b):
    gbuf, dbuf, wbuf, rows, gsem, ssem = b
    pltpu.make_async_copy(table_hbm.at[gbuf], rows, gsem).wait()
    pltpu.make_async_copy(edst_hbm.at[pl.ds(base, _K)], dbuf, gsem).wait()
    pltpu.make_async_copy(w_hbm.at[pl.ds(base, _K)], wbuf, gsem).wait()
    @plsc.parallel_loop(0, _K, unroll=2)
    def scale(e):
      wsp = plsc.load_gather(wbuf, [jnp.full((16,), e, jnp.int32)])
      for j in range(8):
        sl = pl.ds(j * 16, 16)
        rows[e, sl] = rows[e, sl] * wsp
    pltpu.async_copy(rows, agg_sp.at[dbuf], ssem, add=True)

  def wait_scatter(b):
    _, dbuf, _, rows, _, ssem = b
    pltpu.make_async_copy(rows, agg_sp.at[dbuf], ssem).wait()

  prep(0, bufs[0])
  prep(_K, bufs[1])
  prep(2 * _K, bufs[2])
  def ring(k, _):
    o = 3 * k * _K
    scale_scatter(bufs[0])
    scale_scatter(bufs[1])
    wait_scatter(bufs[0])
    prep(o + 3 * _K, bufs[0])
    scale_scatter(bufs[2])
    wait_scatter(bufs[1])
    prep(o + 4 * _K, bufs[1])
    wait_scatter(bufs[2])
    prep(o + 5 * _K, bufs[2])
    return 0
  lax.fori_loop(0, 40, ring, 0)
  scale_scatter(bufs[0])
  scale_scatter(bufs[1])
  wait_scatter(bufs[0])
  prep(123 * _K, bufs[0])
  scale_scatter(bufs[2])
  wait_scatter(bufs[1])
  prep(124 * _K, bufs[1])
  scale_scatter(bufs[0])
  scale_scatter(bufs[1])
  wait_scatter(bufs[2])
  wait_scatter(bufs[0])
  wait_scatter(bufs[1])

  plsc.subcore_barrier()
  pltpu.sync_copy(agg_sp.at[pl.ds(s * 640, 640)],
                  out_hbm.at[c, pl.ds(s * 640, 640)])


_sc_agg = pl.kernel(
    _sc_agg_body,
    out_type=jax.ShapeDtypeStruct((2, _NP, _HC), jnp.float32),
    mesh=_mesh2,
    scratch_types=[
        pltpu.VMEM((_EPS,), jnp.int32),
        pltpu.VMEM((_K,), jnp.int32),
        pltpu.VMEM((_K,), jnp.int32),
        pltpu.VMEM((_K,), jnp.float32),
        pltpu.VMEM((_K, _HC), jnp.float32),
        pltpu.VMEM((_K,), jnp.int32),
        pltpu.VMEM((_K,), jnp.int32),
        pltpu.VMEM((_K,), jnp.float32),
        pltpu.VMEM((_K, _HC), jnp.float32),
        pltpu.VMEM((_K,), jnp.int32),
        pltpu.VMEM((_K,), jnp.int32),
        pltpu.VMEM((_K,), jnp.float32),
        pltpu.VMEM((_K, _HC), jnp.float32),
        pltpu.VMEM_SHARED((_NP, _HC), jnp.float32),
        pltpu.SemaphoreType.DMA,
        pltpu.SemaphoreType.DMA,
        pltpu.SemaphoreType.DMA,
        pltpu.SemaphoreType.DMA,
        pltpu.SemaphoreType.DMA,
        pltpu.SemaphoreType.DMA,
    ],
    compiler_params=pltpu.CompilerParams(needs_layout_passes=False),
)


def _transform_body(x_ref, w_ref, o_ref):
  o_ref[...] = jnp.dot(x_ref[...], w_ref[0],
                       preferred_element_type=jnp.float32)


def _transform(x_pad, Wr):
  return pl.pallas_call(
      _transform_body,
      grid=(_NP // 1024, 2, _R),
      in_specs=[
          pl.BlockSpec((1024, _D), lambda t, c, r: (t, 0)),
          pl.BlockSpec((1, _D, _HC), lambda t, c, r: (r, 0, c)),
      ],
      out_specs=pl.BlockSpec(
          (1024, _HC), lambda t, c, r: (c * (_CF // 1024) + r * 10 + t, 0)),
      out_shape=jax.ShapeDtypeStruct((2 * _CF, _HC), jnp.float32),
  )(x_pad, Wr)


def _combine1_body(x_ref, wr_ref, a_ref, b_ref, o_ref):
  y = jnp.dot(x_ref[...], wr_ref[...], preferred_element_type=jnp.float32)
  agg = jnp.concatenate([a_ref[0], a_ref[1]], axis=1)
  o_ref[...] = jnp.maximum(y + agg + b_ref[...], 0.0)


def _combine2_body(x_ref, wr_ref, a_ref, b_ref, aw_ref, ab_ref, o_ref, l_ref):
  y = jnp.dot(x_ref[...], wr_ref[...], preferred_element_type=jnp.float32)
  agg = jnp.concatenate([a_ref[0], a_ref[1]], axis=1)
  y = y + agg + b_ref[...]
  o_ref[...] = y
  l_ref[...] = jnp.tanh(
      jnp.dot(y, aw_ref[...], preferred_element_type=jnp.float32)
      + ab_ref[...])


def _combine1(x_pad, Wroot, agg, b_row):
  return pl.pallas_call(
      _combine1_body,
      grid=(_NP // 1024,),
      in_specs=[
          pl.BlockSpec((1024, _D), lambda t: (t, 0)),
          pl.BlockSpec((_D, _D), lambda t: (0, 0)),
          pl.BlockSpec((2, 1024, _HC), lambda t: (0, t, 0)),
          pl.BlockSpec((1, _D), lambda t: (0, 0)),
      ],
      out_specs=pl.BlockSpec((1024, _D), lambda t: (t, 0)),
      out_shape=jax.ShapeDtypeStruct((_NP, _D), jnp.float32),
  )(x_pad, Wroot, agg, b_row)


def _combine2(x_pad, Wroot, agg, b_row, aw_p, ab_p):
  return pl.pallas_call(
      _combine2_body,
      grid=(_NP // 1024,),
      in_specs=[
          pl.BlockSpec((1024, _D), lambda t: (t, 0)),
          pl.BlockSpec((_D, _D), lambda t: (0, 0)),
          pl.BlockSpec((2, 1024, _HC), lambda t: (0, t, 0)),
          pl.BlockSpec((1, _D), lambda t: (0, 0)),
          pl.BlockSpec((_D, _HC), lambda t: (0, 0)),
          pl.BlockSpec((1, _HC), lambda t: (0, 0)),
      ],
      out_specs=[
          pl.BlockSpec((1024, _D), lambda t: (t, 0)),
          pl.BlockSpec((1024, _HC), lambda t: (t, 0)),
      ],
      out_shape=[
          jax.ShapeDtypeStruct((_NP, _D), jnp.float32),
          jax.ShapeDtypeStruct((_NP, _HC), jnp.float32),
      ],
  )(x_pad, Wroot, agg, b_row, aw_p, ab_p)


def _head_body(l_ref, m_ref, ne_ref, df_ref, wd_ref, bd_ref, g_ref, be_ref,
               w1_ref, b1_ref, w2_ref, b2_ref, w3_ref, b3_ref,
               logit_ref, attn_ref):
  lrow = jnp.broadcast_to(l_ref[...], (16, _NP))
  cols = lax.broadcasted_iota(jnp.int32, (16, _NP), 1)
  lg = jnp.where(m_ref[...] == 0, -1e9, lrow)
  lg = jnp.where(cols >= _N, -jnp.inf, lg)
  mx = jnp.max(lg, axis=1, keepdims=True)
  ex = jnp.exp(lg - mx)
  sm = jnp.sum(ex, axis=1, keepdims=True)
  attn = ex / sm
  attn_ref[...] = attn
  pg = jnp.dot(attn, ne_ref[...], preferred_element_type=jnp.float32)
  h = jnp.maximum(
      jnp.dot(df_ref[...], wd_ref[...], preferred_element_type=jnp.float32)
      + bd_ref[...], 0.0)
  h = h / jnp.sqrt(1.0 + 1e-5) * g_ref[...] + be_ref[...]
  comb = jnp.concatenate([pg, h], axis=1)
  z = jnp.maximum(
      jnp.dot(comb, w1_ref[...], preferred_element_type=jnp.float32)
      + b1_ref[...], 0.0)
  z = jnp.maximum(
      jnp.dot(z, w2_ref[...], preferred_element_type=jnp.float32)
      + b2_ref[...], 0.0)
  logit_ref[...] = (
      jnp.dot(z, w3_ref[...], preferred_element_type=jnp.float32)
      + b3_ref[...])


def _head(lrow, mask_pad, ne, df_pad, wd_p, bd_p, g_p, be_p,
          w1_p, b1_p, w2_p, b2_p, w3_p, b3_p):
  return pl.pallas_call(
      _head_body,
      out_shape=[
          jax.ShapeDtypeStruct((16, _HC), jnp.float32),
          jax.ShapeDtypeStruct((16, _NP), jnp.float32),
      ],
  )(lrow, mask_pad, ne, df_pad, wd_p, bd_p, g_p, be_p,
    w1_p, b1_p, w2_p, b2_p, w3_p, b3_p)


@jax.jit
def kernel(edge_index, edge_types, patient_masks, direct_features, emb,
           Wr1, Wroot1, b1, Wr2, Wroot2, b2, attn_w, attn_b, Wd, bd,
           gamma, beta, Wc1, bc1, Wc2, bc2, Wc3, bc3):
  esrc = edge_index[0]
  edst = edge_index[1]
  w_e, gidx = _sc_weights(esrc, edst, edge_types)

  emb_pad = jnp.pad(emb, ((0, _NP - _N), (0, 0)))
  t1 = _transform(emb_pad, Wr1)
  agg1 = _sc_agg(t1, gidx, edst, w_e)
  x1 = _combine1(emb_pad, Wroot1, agg1, b1.reshape(1, -1))

  aw_p = jnp.pad(attn_w, ((0, 0), (0, _HC - 1)))
  ab_p = jnp.pad(attn_b.reshape(1, 1), ((0, 0), (0, _HC - 1)))
  t2 = _transform(x1, Wr2)
  agg2 = _sc_agg(t2, gidx, edst, w_e)
  ne, ln = _combine2(x1, Wroot2, agg2, b2.reshape(1, -1), aw_p, ab_p)

  lrow = ln[:, 0].reshape(1, _NP)
  mask_pad = jnp.pad(patient_masks, ((0, 0), (0, _NP - _N)))
  df_pad = jnp.pad(direct_features, ((0, 0), (0, _HC - 11)))
  wd_p = jnp.pad(Wd, ((0, _HC - 11), (0, _HC - 64)))
  bd_p = jnp.pad(bd.reshape(1, -1), ((0, 0), (0, _HC - 64)))
  g_p = jnp.pad(gamma.reshape(1, -1), ((0, 0), (0, _HC - 64)))
  be_p = jnp.pad(beta.reshape(1, -1), ((0, 0), (0, _HC - 64)))
  w1_p = jnp.zeros((_D + _HC, _HC), jnp.float32)
  w1_p = w1_p.at[:_D, :64].set(Wc1[:_D])
  w1_p = w1_p.at[_D:_D + 64, :64].set(Wc1[_D:])
  b1_p = jnp.pad(bc1.reshape(1, -1), ((0, 0), (0, _HC - 64)))
  w2_p = jnp.pad(Wc2, ((0, _HC - 64), (0, _HC - 32)))
  b2_p = jnp.pad(bc2.reshape(1, -1), ((0, 0), (0, _HC - 32)))
  w3_p = jnp.pad(Wc3, ((0, _HC - 32), (0, _HC - 3)))
  b3_p = jnp.pad(bc3.reshape(1, -1), ((0, 0), (0, _HC - 3)))

  logits_p, attn = _head(lrow, mask_pad, ne, df_pad, wd_p, bd_p, g_p, be_p,
                         w1_p, b1_p, w2_p, b2_p, w3_p, b3_p)
  return logits_p[:, :3], attn[:, :_N]
